# Initial kernel scaffold; baseline (speedup 1.0000x reference)
import jax
import jax.numpy as jnp
from jax.experimental import pallas as pl


def kernel(x, memory_bank):
    raise NotImplementedError("write your pallas kernel here")


# speedup vs baseline: 2.2982x; 2.2982x over previous
import math

import jax
import jax.numpy as jnp
from jax.experimental import pallas as pl
from jax.experimental.pallas import tpu as pltpu

_N = 100000
_NPAD = 102400
_D = 128
_K = 1024
_NB = _NPAD // _K
_NQ = 392
_K2 = 1024
_NB2 = _NPAD // _K2
_NUM_NEIGHBORS = 9
_PAD_VAL = 30000.0


def _dot3(q, b):
    dn = (((1,), (1,)), ((), ()))
    qh = q.astype(jnp.bfloat16)
    ql = (q - qh.astype(jnp.float32)).astype(jnp.bfloat16)
    bh = b.astype(jnp.bfloat16)
    bl = (b - bh.astype(jnp.float32)).astype(jnp.bfloat16)
    out = jax.lax.dot_general(qh, bh, dn, preferred_element_type=jnp.float32)
    out = out + jax.lax.dot_general(qh, bl, dn,
                                    preferred_element_type=jnp.float32)
    out = out + jax.lax.dot_general(ql, bh, dn,
                                    preferred_element_type=jnp.float32)
    return out


def _stage1_body(q_ref, bank_ref, bn_ref, dmin_ref, idx_ref):
    j = pl.program_id(1)

    @pl.when(j == 0)
    def _init():
        dmin_ref[...] = jnp.full(dmin_ref.shape, jnp.inf, dmin_ref.dtype)
        idx_ref[...] = jnp.zeros(idx_ref.shape, idx_ref.dtype)

    q = q_ref[...]
    b = bank_ref[...]
    dots = _dot3(q, b)
    part = bn_ref[pl.ds(j, 1), :] - 2.0 * dots
    m = jnp.min(part, axis=1, keepdims=True)
    lane = jax.lax.broadcasted_iota(jnp.int32, part.shape, 1)
    cand = jnp.where(part == m, lane, jnp.int32(2 ** 30))
    a = jnp.min(cand, axis=1, keepdims=True) + j * _K
    prev = dmin_ref[...]
    better = m < prev
    dmin_ref[...] = jnp.where(better, m, prev)
    idx_ref[...] = jnp.where(better, a, idx_ref[...])

    @pl.when(j == _NB - 1)
    def _fin():
        qn = jnp.sum(q * q, axis=1)[:, None]
        d2 = qn + dmin_ref[...]
        dmin_ref[...] = jnp.sqrt(jnp.clip(d2, 1e-12, None))


def _stage1(q, bank, bn):
    nq = q.shape[0]
    nqb = nq // _NQ
    return pl.pallas_call(
        _stage1_body,
        grid=(nqb, _NB),
        in_specs=[
            pl.BlockSpec((_NQ, _D), lambda i, j: (i, 0)),
            pl.BlockSpec((_K, _D), lambda i, j: (j, 0)),
            pl.BlockSpec((_NB, _K), lambda i, j: (0, 0)),
        ],
        out_specs=[
            pl.BlockSpec((_NQ, 1), lambda i, j: (i, 0)),
            pl.BlockSpec((_NQ, 1), lambda i, j: (i, 0)),
        ],
        out_shape=[
            jax.ShapeDtypeStruct((nq, 1), jnp.float32),
            jax.ShapeDtypeStruct((nq, 1), jnp.int32),
        ],
        compiler_params=pltpu.CompilerParams(
            dimension_semantics=("parallel", "arbitrary")),
    )(q, bank, bn)


def _stage2_body(q2_ref, bank_ref, bn_ref, score0_ref, out_ref, scr_ref):
    j = pl.program_id(0)

    q = q2_ref[...]
    b = bank_ref[...]
    qn = jnp.sum(q * q, axis=1)
    dots = _dot3(q, b)
    d2 = qn[:, None] + bn_ref[pl.ds(j, 1), :] - 2.0 * dots
    scr_ref[:, j, :] = d2

    @pl.when(j == _NB2 - 1)
    def _fin():
        val = scr_ref[...]
        dsel = val[0:2]
        dval = val[2:4]
        li = (jax.lax.broadcasted_iota(jnp.int32, (_NB2, _K2), 0) * _K2
              + jax.lax.broadcasted_iota(jnp.int32, (_NB2, _K2), 1))
        li = li[None]
        big = jnp.int32(2 ** 30)
        cur = dsel
        sume = jnp.zeros((2, 1), jnp.float32)
        for _ in range(_NUM_NEIGHBORS):
            m = jnp.min(jnp.min(cur, axis=2, keepdims=True), axis=1,
                        keepdims=True)
            ismin = cur <= m
            istar = jnp.min(jnp.min(jnp.where(ismin, li, big), axis=2,
                                    keepdims=True), axis=1, keepdims=True)
            onehot = li == istar
            v = jnp.sum(jnp.sum(jnp.where(onehot, dval, 0.0), axis=2),
                        axis=1)
            dist = jnp.sqrt(jnp.clip(v, 1e-12, None))
            sume = sume + jnp.exp(dist)[:, None]
            cur = jnp.where(onehot, jnp.inf, cur)
        score0 = score0_ref[...]
        w = 1.0 - jnp.exp(score0) / sume
        out_ref[...] = w * score0


def _stage2(q2, bank, bn, score0):
    return pl.pallas_call(
        _stage2_body,
        grid=(_NB2,),
        in_specs=[
            pl.BlockSpec((8, _D), lambda j: (0, 0)),
            pl.BlockSpec((_K2, _D), lambda j: (j, 0)),
            pl.BlockSpec((_NB2, _K2), lambda j: (0, 0)),
            pl.BlockSpec((2, 1), lambda j: (0, 0)),
        ],
        out_specs=pl.BlockSpec((2, 1), lambda j: (0, 0)),
        out_shape=jax.ShapeDtypeStruct((2, 1), jnp.float32),
        scratch_shapes=[pltpu.VMEM((8, _NB2, _K2), jnp.float32)],
        compiler_params=pltpu.CompilerParams(
            dimension_semantics=("arbitrary",)),
    )(q2, bank, bn, score0)


def kernel(x, memory_bank):
    B, P, D = x.shape
    q = x.reshape(B * P, D)
    bank_pad = jnp.concatenate(
        [memory_bank,
         jnp.full((_NPAD - _N, D), _PAD_VAL, memory_bank.dtype)], axis=0)
    bn = jnp.sum(bank_pad * bank_pad, axis=1).reshape(_NB, _K)
    dmin, idx = _stage1(q, bank_pad, bn)

    patch_scores = dmin[:, 0].reshape(B, P)
    locations = idx[:, 0].reshape(B, P)
    side = math.isqrt(P)
    anomaly_map = patch_scores.reshape(B, side, side)

    max_idx = jnp.argmax(patch_scores, axis=1)
    score0 = jnp.take_along_axis(patch_scores, max_idx[:, None], axis=1)
    nn_idx = jnp.take_along_axis(locations, max_idx[:, None], axis=1)[:, 0]
    max_feat = jnp.take_along_axis(x, max_idx[:, None, None], axis=1)[:, 0, :]
    nn_sample = jnp.take(memory_bank, nn_idx, axis=0)
    q2 = jnp.concatenate(
        [nn_sample, max_feat, jnp.zeros((4, D), jnp.float32)], axis=0)

    score = _stage2(q2, bank_pad, bn, score0)
    return anomaly_map, score[:, 0]


# speedup vs baseline: 2.7200x; 1.1836x over previous
import math

import jax
import jax.numpy as jnp
from jax.experimental import pallas as pl
from jax.experimental.pallas import tpu as pltpu

_N = 100000
_NPAD = 102400
_D = 128
_K = 1024
_NB = _NPAD // _K
_NQ = 784
_CD = 512
_K2 = 1024
_NB2 = _NPAD // _K2
_NUM_NEIGHBORS = 9
_PAD_VAL = 30000.0


def _dot3(q, b):
    dn = (((1,), (1,)), ((), ()))
    qh = q.astype(jnp.bfloat16)
    ql = (q - qh.astype(jnp.float32)).astype(jnp.bfloat16)
    bh = b.astype(jnp.bfloat16)
    bl = (b - bh.astype(jnp.float32)).astype(jnp.bfloat16)
    out = jax.lax.dot_general(qh, bh, dn, preferred_element_type=jnp.float32)
    out = out + jax.lax.dot_general(qh, bl, dn,
                                    preferred_element_type=jnp.float32)
    out = out + jax.lax.dot_general(ql, bh, dn,
                                    preferred_element_type=jnp.float32)
    return out


def _stage1_body(qc_ref, qn_ref, bank_ref, bn2_ref, dmin_ref, idx_ref):
    j = pl.program_id(1)

    @pl.when(j == 0)
    def _init():
        dmin_ref[...] = jnp.full(dmin_ref.shape, jnp.inf, dmin_ref.dtype)
        idx_ref[...] = jnp.zeros(idx_ref.shape, idx_ref.dtype)

    b = bank_ref[...]
    bh = b.astype(jnp.bfloat16)
    bl = (b - bh.astype(jnp.float32)).astype(jnp.bfloat16)
    bn2 = bn2_ref[...]
    zpad = jnp.zeros((_K, _CD - 3 * _D - 2), jnp.bfloat16)
    bcat = jnp.concatenate([bh, bl, bh, bn2, zpad], axis=1)
    dn = (((1,), (1,)), ((), ()))
    part = jax.lax.dot_general(qc_ref[...], bcat, dn,
                               preferred_element_type=jnp.float32)
    m = jnp.min(part, axis=1, keepdims=True)
    lane = jax.lax.broadcasted_iota(jnp.int32, part.shape, 1)
    cand = jnp.where(part == m, lane, jnp.int32(2 ** 30))
    a = jnp.min(cand, axis=1, keepdims=True) + j * _K
    prev = dmin_ref[...]
    better = m < prev
    dmin_ref[...] = jnp.where(better, m, prev)
    idx_ref[...] = jnp.where(better, a, idx_ref[...])

    @pl.when(j == _NB - 1)
    def _fin():
        d2 = qn_ref[...] + dmin_ref[...]
        dmin_ref[...] = jnp.sqrt(jnp.clip(d2, 1e-12, None))


def _stage1(qc, qn, bank, bn2):
    nq = qc.shape[0]
    nqb = nq // _NQ
    return pl.pallas_call(
        _stage1_body,
        grid=(nqb, _NB),
        in_specs=[
            pl.BlockSpec((_NQ, _CD), lambda i, j: (i, 0)),
            pl.BlockSpec((_NQ, 1), lambda i, j: (i, 0)),
            pl.BlockSpec((_K, _D), lambda i, j: (j, 0)),
            pl.BlockSpec((_K, 2), lambda i, j: (j, 0)),
        ],
        out_specs=[
            pl.BlockSpec((_NQ, 1), lambda i, j: (i, 0)),
            pl.BlockSpec((_NQ, 1), lambda i, j: (i, 0)),
        ],
        out_shape=[
            jax.ShapeDtypeStruct((nq, 1), jnp.float32),
            jax.ShapeDtypeStruct((nq, 1), jnp.int32),
        ],
        compiler_params=pltpu.CompilerParams(
            dimension_semantics=("parallel", "arbitrary")),
    )(qc, qn, bank, bn2)


def _stage2_body(q2_ref, bank_ref, bn_ref, score0_ref, out_ref, scr_ref):
    j = pl.program_id(0)

    q = q2_ref[...]
    b = bank_ref[...]
    qn = jnp.sum(q * q, axis=1)
    dots = _dot3(q, b)
    d2 = qn[:, None] + bn_ref[pl.ds(j, 1), :] - 2.0 * dots
    scr_ref[:, j, :] = d2

    @pl.when(j == _NB2 - 1)
    def _fin():
        val = scr_ref[...]
        dsel = val[0:2]
        dval = val[2:4]
        li = (jax.lax.broadcasted_iota(jnp.int32, (_NB2, _K2), 0) * _K2
              + jax.lax.broadcasted_iota(jnp.int32, (_NB2, _K2), 1))
        li = li[None]
        big = jnp.int32(2 ** 30)
        cur = dsel
        sume = jnp.zeros((2, 1), jnp.float32)
        for _ in range(_NUM_NEIGHBORS):
            m = jnp.min(jnp.min(cur, axis=2, keepdims=True), axis=1,
                        keepdims=True)
            ismin = cur <= m
            istar = jnp.min(jnp.min(jnp.where(ismin, li, big), axis=2,
                                    keepdims=True), axis=1, keepdims=True)
            onehot = li == istar
            v = jnp.sum(jnp.sum(jnp.where(onehot, dval, 0.0), axis=2),
                        axis=1)
            dist = jnp.sqrt(jnp.clip(v, 1e-12, None))
            sume = sume + jnp.exp(dist)[:, None]
            cur = jnp.where(onehot, jnp.inf, cur)
        score0 = score0_ref[...]
        w = 1.0 - jnp.exp(score0) / sume
        out_ref[...] = w * score0


def _stage2(q2, bank, bn, score0):
    return pl.pallas_call(
        _stage2_body,
        grid=(_NB2,),
        in_specs=[
            pl.BlockSpec((8, _D), lambda j: (0, 0)),
            pl.BlockSpec((_K2, _D), lambda j: (j, 0)),
            pl.BlockSpec((_NB2, _K2), lambda j: (0, 0)),
            pl.BlockSpec((2, 1), lambda j: (0, 0)),
        ],
        out_specs=pl.BlockSpec((2, 1), lambda j: (0, 0)),
        out_shape=jax.ShapeDtypeStruct((2, 1), jnp.float32),
        scratch_shapes=[pltpu.VMEM((8, _NB2, _K2), jnp.float32)],
        compiler_params=pltpu.CompilerParams(
            dimension_semantics=("arbitrary",)),
    )(q2, bank, bn, score0)


def kernel(x, memory_bank):
    B, P, D = x.shape
    q = x.reshape(B * P, D)
    bank_pad = jnp.concatenate(
        [memory_bank,
         jnp.full((_NPAD - _N, D), _PAD_VAL, memory_bank.dtype)], axis=0)
    bn_flat = jnp.sum(bank_pad * bank_pad, axis=1)
    bn = bn_flat.reshape(_NB, _K)
    bnh = bn_flat.astype(jnp.bfloat16)
    bnl = (bn_flat - bnh.astype(jnp.float32)).astype(jnp.bfloat16)
    bn2 = jnp.stack([bnh, bnl], axis=1)

    s = -2.0 * q
    sh = s.astype(jnp.bfloat16)
    sl = (s - sh.astype(jnp.float32)).astype(jnp.bfloat16)
    nq = q.shape[0]
    qc = jnp.concatenate(
        [sh, sh, sl, jnp.ones((nq, 2), jnp.bfloat16),
         jnp.zeros((nq, _CD - 3 * _D - 2), jnp.bfloat16)], axis=1)
    qn = jnp.sum(q * q, axis=1, keepdims=True)
    dmin, idx = _stage1(qc, qn, bank_pad, bn2)

    patch_scores = dmin[:, 0].reshape(B, P)
    locations = idx[:, 0].reshape(B, P)
    side = math.isqrt(P)
    anomaly_map = patch_scores.reshape(B, side, side)

    max_idx = jnp.argmax(patch_scores, axis=1)
    score0 = jnp.take_along_axis(patch_scores, max_idx[:, None], axis=1)
    nn_idx = jnp.take_along_axis(locations, max_idx[:, None], axis=1)[:, 0]
    max_feat = jnp.take_along_axis(x, max_idx[:, None, None], axis=1)[:, 0, :]
    nn_sample = jnp.take(memory_bank, nn_idx, axis=0)
    q2 = jnp.concatenate(
        [nn_sample, max_feat, jnp.zeros((4, D), jnp.float32)], axis=0)

    score = _stage2(q2, bank_pad, bn, score0)
    return anomaly_map, score[:, 0]


# speedup vs baseline: 3.1960x; 1.1750x over previous
import math

import jax
import jax.numpy as jnp
from jax.experimental import pallas as pl
from jax.experimental.pallas import tpu as pltpu

_N = 100000
_NPAD = 102400
_D = 128
_K = 1024
_NB = _NPAD // _K
_NQ = 1568
_CD = 512
_K2 = 1024
_NB2 = _NPAD // _K2
_NUM_NEIGHBORS = 9
_PAD_VAL = 30000.0


def _dot3(q, b):
    dn = (((1,), (1,)), ((), ()))
    qh = q.astype(jnp.bfloat16)
    ql = (q - qh.astype(jnp.float32)).astype(jnp.bfloat16)
    bh = b.astype(jnp.bfloat16)
    bl = (b - bh.astype(jnp.float32)).astype(jnp.bfloat16)
    out = jax.lax.dot_general(qh, bh, dn, preferred_element_type=jnp.float32)
    out = out + jax.lax.dot_general(qh, bl, dn,
                                    preferred_element_type=jnp.float32)
    out = out + jax.lax.dot_general(ql, bh, dn,
                                    preferred_element_type=jnp.float32)
    return out


def _stage1_body(qc_ref, qn_ref, bank_ref, bn2_ref, dmin_ref):
    j = pl.program_id(1)

    @pl.when(j == 0)
    def _init():
        dmin_ref[...] = jnp.full(dmin_ref.shape, jnp.inf, dmin_ref.dtype)

    b = bank_ref[...]
    bh = b.astype(jnp.bfloat16)
    bl = (b - bh.astype(jnp.float32)).astype(jnp.bfloat16)
    bn2 = bn2_ref[...]
    zpad = jnp.zeros((_K, _CD - 3 * _D - 2), jnp.bfloat16)
    bcat = jnp.concatenate([bh, bl, bh, bn2, zpad], axis=1)
    dn = (((1,), (1,)), ((), ()))
    part = jax.lax.dot_general(qc_ref[...], bcat, dn,
                               preferred_element_type=jnp.float32)
    m = jnp.min(part, axis=1, keepdims=True)
    dmin_ref[...] = jnp.minimum(dmin_ref[...], m)

    @pl.when(j == _NB - 1)
    def _fin():
        d2 = qn_ref[...] + dmin_ref[...]
        dmin_ref[...] = jnp.sqrt(jnp.clip(d2, 1e-12, None))


def _stage1(qc, qn, bank, bn2):
    nq = qc.shape[0]
    nqb = nq // _NQ
    return pl.pallas_call(
        _stage1_body,
        grid=(nqb, _NB),
        in_specs=[
            pl.BlockSpec((_NQ, _CD), lambda i, j: (i, 0)),
            pl.BlockSpec((_NQ, 1), lambda i, j: (i, 0)),
            pl.BlockSpec((_K, _D), lambda i, j: (j, 0)),
            pl.BlockSpec((_K, 2), lambda i, j: (j, 0)),
        ],
        out_specs=pl.BlockSpec((_NQ, 1), lambda i, j: (i, 0)),
        out_shape=jax.ShapeDtypeStruct((nq, 1), jnp.float32),
        compiler_params=pltpu.CompilerParams(
            dimension_semantics=("parallel", "arbitrary")),
    )(qc, qn, bank, bn2)


def _stage2_body(q2_ref, bank_ref, bn_ref, score0_ref, out_ref,
                 dval_ref, dsel_ref, best_ref, nnf_ref):
    p = pl.program_id(0)
    j = pl.program_id(1)
    dn = (((1,), (1,)), ((), ()))

    b = bank_ref[...]
    bnrow = bn_ref[pl.ds(j, 1), :]

    @pl.when((p == 0) & (j == 0))
    def _init():
        best_ref[...] = jnp.full(best_ref.shape, jnp.inf, best_ref.dtype)
        nnf_ref[...] = jnp.zeros(nnf_ref.shape, nnf_ref.dtype)

    @pl.when(p == 0)
    def _pass0():
        q = q2_ref[...]
        qn = jnp.sum(q * q, axis=1)
        d2 = qn[:, None] + bnrow - 2.0 * _dot3(q, b)
        dval_ref[:, j, :] = d2[0:2]
        m = jnp.min(d2, axis=1, keepdims=True)
        lane = jax.lax.broadcasted_iota(jnp.int32, d2.shape, 1)
        il = jnp.min(jnp.where(d2 == m, lane, jnp.int32(2 ** 30)),
                     axis=1, keepdims=True)
        oh = (lane == il).astype(jnp.bfloat16)
        bh = b.astype(jnp.bfloat16)
        bl = (b - bh.astype(jnp.float32)).astype(jnp.bfloat16)
        dn2 = (((1,), (0,)), ((), ()))
        e = (jax.lax.dot_general(oh, bh, dn2,
                                 preferred_element_type=jnp.float32)
             + jax.lax.dot_general(oh, bl, dn2,
                                   preferred_element_type=jnp.float32))
        better = m < best_ref[...]
        best_ref[...] = jnp.where(better, m, best_ref[...])
        nnf_ref[...] = jnp.where(better, e, nnf_ref[...])

    @pl.when(p == 1)
    def _pass1():
        q = nnf_ref[...]
        qn = jnp.sum(q * q, axis=1)
        d2 = qn[:, None] + bnrow - 2.0 * _dot3(q, b)
        dsel_ref[:, j, :] = d2[0:2]

    @pl.when((p == 1) & (j == _NB2 - 1))
    def _fin():
        dsel = dsel_ref[...]
        dval = dval_ref[...]
        li = (jax.lax.broadcasted_iota(jnp.int32, (_NB2, _K2), 0) * _K2
              + jax.lax.broadcasted_iota(jnp.int32, (_NB2, _K2), 1))
        li = li[None]
        big = jnp.int32(2 ** 30)
        cur = dsel
        sume = jnp.zeros((2, 1), jnp.float32)
        for _ in range(_NUM_NEIGHBORS):
            m = jnp.min(jnp.min(cur, axis=2, keepdims=True), axis=1,
                        keepdims=True)
            ismin = cur <= m
            istar = jnp.min(jnp.min(jnp.where(ismin, li, big), axis=2,
                                    keepdims=True), axis=1, keepdims=True)
            onehot = li == istar
            v = jnp.sum(jnp.sum(jnp.where(onehot, dval, 0.0), axis=2),
                        axis=1)
            dist = jnp.sqrt(jnp.clip(v, 1e-12, None))
            sume = sume + jnp.exp(dist)[:, None]
            cur = jnp.where(onehot, jnp.inf, cur)
        score0 = score0_ref[...]
        w = 1.0 - jnp.exp(score0) / sume
        out_ref[...] = w * score0


def _stage2(q2, bank, bn, score0):
    return pl.pallas_call(
        _stage2_body,
        grid=(2, _NB2),
        in_specs=[
            pl.BlockSpec((8, _D), lambda p, j: (0, 0)),
            pl.BlockSpec((_K2, _D), lambda p, j: (j, 0)),
            pl.BlockSpec((_NB2, _K2), lambda p, j: (0, 0)),
            pl.BlockSpec((2, 1), lambda p, j: (0, 0)),
        ],
        out_specs=pl.BlockSpec((2, 1), lambda p, j: (0, 0)),
        out_shape=jax.ShapeDtypeStruct((2, 1), jnp.float32),
        scratch_shapes=[
            pltpu.VMEM((2, _NB2, _K2), jnp.float32),
            pltpu.VMEM((2, _NB2, _K2), jnp.float32),
            pltpu.VMEM((8, 1), jnp.float32),
            pltpu.VMEM((8, _D), jnp.float32),
        ],
        compiler_params=pltpu.CompilerParams(
            dimension_semantics=("arbitrary", "arbitrary")),
    )(q2, bank, bn, score0)


def kernel(x, memory_bank):
    B, P, D = x.shape
    q = x.reshape(B * P, D)
    bank_pad = jnp.concatenate(
        [memory_bank,
         jnp.full((_NPAD - _N, D), _PAD_VAL, memory_bank.dtype)], axis=0)
    bn_flat = jnp.sum(bank_pad * bank_pad, axis=1)
    bn = bn_flat.reshape(_NB, _K)
    bnh = bn_flat.astype(jnp.bfloat16)
    bnl = (bn_flat - bnh.astype(jnp.float32)).astype(jnp.bfloat16)
    bn2 = jnp.stack([bnh, bnl], axis=1)

    s = -2.0 * q
    sh = s.astype(jnp.bfloat16)
    sl = (s - sh.astype(jnp.float32)).astype(jnp.bfloat16)
    nq = q.shape[0]
    qc = jnp.concatenate(
        [sh, sh, sl, jnp.ones((nq, 2), jnp.bfloat16),
         jnp.zeros((nq, _CD - 3 * _D - 2), jnp.bfloat16)], axis=1)
    qn = jnp.sum(q * q, axis=1, keepdims=True)
    dmin = _stage1(qc, qn, bank_pad, bn2)

    patch_scores = dmin[:, 0].reshape(B, P)
    side = math.isqrt(P)
    anomaly_map = patch_scores.reshape(B, side, side)

    max_idx = jnp.argmax(patch_scores, axis=1)
    score0 = jnp.take_along_axis(patch_scores, max_idx[:, None], axis=1)
    max_feat = jnp.take_along_axis(x, max_idx[:, None, None], axis=1)[:, 0, :]
    q2 = jnp.concatenate([max_feat, jnp.zeros((6, D), jnp.float32)], axis=0)

    score = _stage2(q2, bank_pad, bn, score0)
    return anomaly_map, score[:, 0]
